# baseline (device time: 239465 ns/iter reference)
import jax
import jax.numpy as jnp
from jax import lax
from jax.experimental import pallas as pl
from jax.experimental.pallas import tpu as pltpu

T = 2048
D = 4096
V_SHARD = 8192
VB = 512
NB = V_SHARD // VB
NEG = -1e30


def kernel(x, W, labels):
    x = x.astype(jnp.bfloat16)
    labels2d = labels.reshape(T, 1)

    def body(x_ref, w_ref, lab_ref, out_ref,
             m_ref, s_ref, lg_ref,
             comm_send, comm_recv, send_sem, recv_sem):
        i = pl.program_id(0)
        my_x = lax.axis_index("x")
        my_y = lax.axis_index("y")

        @pl.when(i == 0)
        def _init():
            m_ref[...] = jnp.full((T, 1), NEG, jnp.float32)
            s_ref[...] = jnp.zeros((T, 1), jnp.float32)
            lg_ref[...] = jnp.zeros((T, 1), jnp.float32)

        wb = w_ref[...].astype(jnp.bfloat16)
        logits = jnp.dot(x_ref[...], wb, preferred_element_type=jnp.float32)

        m_old = m_ref[...]
        m_new = jnp.maximum(m_old, jnp.max(logits, axis=1, keepdims=True))
        s_ref[...] = (s_ref[...] * jnp.exp(m_old - m_new)
                      + jnp.sum(jnp.exp(logits - m_new), axis=1, keepdims=True))
        m_ref[...] = m_new

        v0 = my_x * V_SHARD + i * VB
        col = lax.broadcasted_iota(jnp.int32, (T, VB), 1)
        hit = col == (lab_ref[...] - v0)
        lg_ref[...] = lg_ref[...] + jnp.sum(
            jnp.where(hit, logits, 0.0), axis=1, keepdims=True)

        @pl.when(i == NB - 1)
        def _exchange():
            comm_send[:, 0:1] = m_ref[...]
            comm_send[:, 1:2] = s_ref[...]
            comm_send[:, 2:3] = lg_ref[...]
            comm_send[:, 3:4] = jnp.zeros((T, 1), jnp.float32)

            partner = (1 - my_x, my_y)
            barrier_sem = pltpu.get_barrier_semaphore()
            pl.semaphore_signal(barrier_sem, inc=1, device_id=partner,
                                device_id_type=pl.DeviceIdType.MESH)
            pl.semaphore_wait(barrier_sem, 1)

            rdma = pltpu.make_async_remote_copy(
                src_ref=comm_send, dst_ref=comm_recv,
                send_sem=send_sem, recv_sem=recv_sem,
                device_id=partner, device_id_type=pl.DeviceIdType.MESH)
            rdma.start()
            rdma.wait()

            m_o = comm_recv[:, 0:1]
            s_o = comm_recv[:, 1:2]
            lg_o = comm_recv[:, 2:3]
            m_tot = jnp.maximum(m_ref[...], m_o)
            s_tot = (s_ref[...] * jnp.exp(m_ref[...] - m_tot)
                     + s_o * jnp.exp(m_o - m_tot))
            out_ref[...] = m_tot + jnp.log(s_tot) - (lg_ref[...] + lg_o)

    out = pl.pallas_call(
        body,
        grid=(NB,),
        in_specs=[
            pl.BlockSpec((T, D), lambda i: (0, 0)),
            pl.BlockSpec((D, VB), lambda i: (0, i)),
            pl.BlockSpec((T, 1), lambda i: (0, 0)),
        ],
        out_specs=pl.BlockSpec((T, 1), lambda i: (0, 0)),
        out_shape=jax.ShapeDtypeStruct((T, 1), jnp.float32),
        scratch_shapes=[
            pltpu.VMEM((T, 1), jnp.float32),
            pltpu.VMEM((T, 1), jnp.float32),
            pltpu.VMEM((T, 1), jnp.float32),
            pltpu.VMEM((T, 4), jnp.float32),
            pltpu.VMEM((T, 4), jnp.float32),
            pltpu.SemaphoreType.DMA,
            pltpu.SemaphoreType.DMA,
        ],
        compiler_params=pltpu.CompilerParams(
            dimension_semantics=("arbitrary",),
            collective_id=0,
        ),
    )(x, W, labels2d)
    return out[:, 0]
